# baseline (speedup 1.0000x reference)
import functools

import jax
import jax.numpy as jnp
from jax import lax
from jax.experimental import pallas as pl
from jax.experimental.pallas import tpu as pltpu
from jax.experimental.pallas import tpu_sc as plsc

_CH = 64
_EPS = 1e-5
_L = 16
_NC = 2
_NS = 16
_NW = _NC * _NS
_BLK = 128
_ACOL = 512
_BB = 128


def _rsqrt(t):
    i = lax.bitcast_convert_type(t, jnp.int32)
    i = jnp.int32(0x5F3759DF) - (i >> 1)
    y = lax.bitcast_convert_type(i, jnp.float32)
    for _ in range(3):
        y = y * (1.5 - 0.5 * t * y * y)
    return y


def _norm_block(src, dst, width, ga, be):
    iot = lax.iota(jnp.int32, _L)

    @plsc.parallel_loop(0, width, _L)
    def _(l0):
        s0 = jnp.zeros((_L,), jnp.float32)
        s1 = jnp.zeros((_L,), jnp.float32)
        q0 = jnp.zeros((_L,), jnp.float32)
        q1 = jnp.zeros((_L,), jnp.float32)
        for c in range(0, _CH, 2):
            v0 = src[c, pl.ds(l0, _L)]
            v1 = src[c + 1, pl.ds(l0, _L)]
            s0 = s0 + v0
            q0 = q0 + v0 * v0
            s1 = s1 + v1
            q1 = q1 + v1 * v1
        s = s0 + s1
        q = q0 + q1
        mean = s * (1.0 / _CH)
        var = q * (1.0 / _CH) - mean * mean
        a = _rsqrt(var + _EPS)
        lvec = l0 + iot
        p2 = lvec >> 2
        q2 = (lvec & 3) * 32
        for w in range(32):
            c1 = w + (0 if w < 16 else _L)
            c2 = c1 + _L
            o1 = (src[c1, pl.ds(l0, _L)] - mean) * a * ga[c1] + be[c1]
            o2 = (src[c2, pl.ds(l0, _L)] - mean) * a * ga[c2] + be[c2]
            wi = plsc.bitcast(
                plsc.pack(o1, o2, format=plsc.PackFormat.INTERLEAVED),
                jnp.int32)
            plsc.store_scatter(dst, [p2, q2 + w], wi)


def _phase_a(n_rows, tabt_hbm, gamma_hbm, beta_hbm, tail_hbm, scr_hbm,
             in0_v, in1_v, out0_v, out1_v, gam_v, bet_v,
             gsem0, gsem1, ssem0, ssem1):
    cid = lax.axis_index("c")
    sid = lax.axis_index("s")
    wid = sid * _NC + cid

    pltpu.sync_copy(gamma_hbm, gam_v)
    pltpu.sync_copy(beta_hbm, bet_v)
    ga, be = [], []
    for k in range(_CH // _L):
        gk = gam_v[pl.ds(k * _L, _L)]
        bk = bet_v[pl.ds(k * _L, _L)]
        for l in range(_L):
            ga.append(gk[l])
            be.append(bk[l])

    n_full = n_rows // _ACOL
    _NB = 2
    n_iter = -((n_full + _NW - 1) // _NW // -_NB) * _NB
    inb = (in0_v, in1_v)
    outb = (out0_v, out1_v)
    gsem = (gsem0, gsem1)
    ssem = (ssem0, ssem1)

    def blk_of(i):
        return wid + i * _NW

    def start_in(i, b):
        pltpu.make_async_copy(
            tabt_hbm.at[:, pl.ds(blk_of(i) * _ACOL, _ACOL)], inb[b],
            gsem[b]).start()

    def wait_in(b):
        pltpu.make_async_copy(
            tabt_hbm.at[:, pl.ds(0, _ACOL)], inb[b], gsem[b]).wait()

    def start_out(i, b):
        pltpu.make_async_copy(
            outb[b], scr_hbm.at[pl.ds(blk_of(i) * (_ACOL // 4), _ACOL // 4)],
            ssem[b]).start()

    def wait_out(b):
        pltpu.make_async_copy(
            outb[b], scr_hbm.at[pl.ds(0, _ACOL // 4)], ssem[b]).wait()

    for b in range(_NB):
        start_in(b, b)

    @pl.loop(0, n_iter, step=_NB)
    def _(ii):
        for b in range(_NB):
            i = ii + b

            @pl.when(blk_of(i) < n_full)
            def _():
                wait_in(b)

                @pl.when(i >= _NB)
                def _():
                    wait_out(b)

                _norm_block(inb[b], outb[b], _ACOL, ga, be)
                start_out(i, b)

                @pl.when(blk_of(i + _NB) < n_full)
                def _():
                    start_in(i + _NB, b)

    for b in range(_NB):
        wait_out(b)

    tail = n_rows - n_full * _ACOL
    if tail:
        @pl.when(wid == _NW - 1)
        def _():
            pltpu.sync_copy(tail_hbm, in0_v.at[:, pl.ds(0, _BLK)])
            _norm_block(in0_v, out0_v, tail, ga, be)
            pltpu.sync_copy(
                out0_v.at[pl.ds(0, tail // 4)],
                scr_hbm.at[pl.ds(n_full * (_ACOL // 4), tail // 4)])


def _phase_b(n_m, xt_hbm, scr_hbm, out_hbm,
             idx_v, in0_v, in1_v, in2_v, in3_v, tr0_v, tr1_v,
             gsem0, gsem1, gsem2, gsem3, ssem0, ssem1):
    cid = lax.axis_index("c")
    sid = lax.axis_index("s")
    wid = sid * _NC + cid

    pltpu.sync_copy(xt_hbm.at[:, pl.ds(wid * _BB, _BB)], idx_v)

    inb = (in0_v, in1_v, in2_v, in3_v)
    trb = (tr0_v, tr1_v)
    gsem = (gsem0, gsem1, gsem2, gsem3)
    ssem = (ssem0, ssem1)
    _NB = 4

    def start_gather(m, b):
        pltpu.make_async_copy(
            scr_hbm.at[idx_v.at[m]], inb[b], gsem[b]).start()

    def wait_gather(b):
        pltpu.make_async_copy(
            scr_hbm.at[idx_v.at[0]], inb[b], gsem[b]).wait()

    def start_store(m, b):
        pltpu.make_async_copy(trb[b], out_hbm.at[m, :, wid], ssem[b]).start()

    def wait_store(b):
        pltpu.make_async_copy(trb[b], out_hbm.at[0, :, wid], ssem[b]).wait()

    for b in range(_NB):
        start_gather(b, b)

    @pl.loop(0, n_m, step=_NB)
    def _(mm):
        for b in range(_NB):
            m = mm + b
            wait_gather(b)

            @pl.when(m >= 2)
            def _():
                wait_store(b % 2)

            src, dst = inb[b], trb[b % 2]

            @plsc.parallel_loop(0, _BB, _L)
            def _(r0):
                rows = r0 + lax.iota(jnp.int32, _L)
                for w0 in range(0, 32, 8):
                    vs = [plsc.load_gather(
                              src, [rows, jnp.full((_L,), w0 + j, jnp.int32)])
                          for j in range(8)]
                    for j in range(8):
                        w = w0 + j
                        lo, hi = plsc.unpack(
                            plsc.bitcast(vs[j], jnp.bfloat16),
                            format=plsc.PackFormat.INTERLEAVED)
                        c1 = w + (0 if w < _L else _L)
                        c2 = c1 + _L
                        dst[c1 // 8, c1 % 8, pl.ds(r0, _L)] = lo
                        dst[c2 // 8, c2 % 8, pl.ds(r0, _L)] = hi

            start_store(m, b % 2)

            @pl.when(m + _NB < n_m)
            def _():
                start_gather(m + _NB, b)

    wait_store(0)
    wait_store(1)


def kernel(x, table, gamma, beta):
    n_b, n_m = x.shape
    n_rows = table.shape[0]
    xt = x.T
    tabt = table.T
    n_full = n_rows // _BLK
    tail_w = n_rows - n_full * _BLK
    tail = jnp.pad(table[n_full * _BLK:].T, ((0, 0), (0, _BLK - tail_w)))

    mesh = plsc.VectorSubcoreMesh(
        core_axis_name="c", subcore_axis_name="s",
        num_cores=_NC, num_subcores=_NS)

    norm_tab = pl.kernel(
        functools.partial(_phase_a, n_rows),
        out_type=jax.ShapeDtypeStruct((n_rows // 4, _BB), jnp.int32),
        mesh=mesh,
        compiler_params=pltpu.CompilerParams(needs_layout_passes=False),
        scratch_types=(
            [pltpu.VMEM((_CH, _ACOL), jnp.float32)] * 2
            + [pltpu.VMEM((_ACOL // 4, _BB), jnp.int32)] * 2
            + [pltpu.VMEM((_CH,), jnp.float32)] * 2
            + [pltpu.SemaphoreType.DMA] * 4
        ),
    )(tabt, gamma, beta, tail)

    scr = norm_tab.reshape(n_rows, 32)

    out5d = pl.kernel(
        functools.partial(_phase_b, n_m),
        out_type=jax.ShapeDtypeStruct((n_m, 8, _NW, 8, _BB), jnp.float32),
        mesh=mesh,
        compiler_params=pltpu.CompilerParams(
            needs_layout_passes=False, use_tc_tiling_on_sc=False),
        scratch_types=(
            [pltpu.VMEM((n_m, _BB), jnp.int32)]
            + [pltpu.VMEM((_BB, 32), jnp.int32)] * 4
            + [pltpu.VMEM((8, 8, _BB), jnp.float32)] * 2
            + [pltpu.SemaphoreType.DMA] * 6
        ),
    )(xt, scr)

    out = out5d.transpose(2, 4, 0, 1, 3).reshape(n_b, n_m, _CH)
    return out


# speedup vs baseline: 1.1858x; 1.1858x over previous
import functools

import jax
import jax.numpy as jnp
from jax import lax
from jax.experimental import pallas as pl
from jax.experimental.pallas import tpu as pltpu
from jax.experimental.pallas import tpu_sc as plsc

_CH = 64
_EPS = 1e-5
_L = 16
_NC = 2
_NS = 16
_NW = _NC * _NS
_BLK = 128
_ACOL = 512
_BB = 128


def _rsqrt(t):
    i = lax.bitcast_convert_type(t, jnp.int32)
    i = jnp.int32(0x5F3759DF) - (i >> 1)
    y = lax.bitcast_convert_type(i, jnp.float32)
    for _ in range(3):
        y = y * (1.5 - 0.5 * t * y * y)
    return y


def _norm_block(src, dst, width, ga, be):
    iot = lax.iota(jnp.int32, _L)

    @plsc.parallel_loop(0, width, _L)
    def _(l0):
        s = jnp.zeros((_L,), jnp.float32)
        q = jnp.zeros((_L,), jnp.float32)
        for c in range(_CH):
            v = src[c, pl.ds(l0, _L)]
            s = s + v
            q = q + v * v
        mean = s * (1.0 / _CH)
        var = q * (1.0 / _CH) - mean * mean
        a = _rsqrt(var + _EPS)
        lvec = l0 + iot
        p2 = lvec >> 2
        q2 = (lvec & 3) * 32
        for w in range(32):
            c1 = w + (0 if w < 16 else _L)
            c2 = c1 + _L
            o1 = (src[c1, pl.ds(l0, _L)] - mean) * a * ga[c1] + be[c1]
            o2 = (src[c2, pl.ds(l0, _L)] - mean) * a * ga[c2] + be[c2]
            wi = plsc.bitcast(
                plsc.pack(o1, o2, format=plsc.PackFormat.INTERLEAVED),
                jnp.int32)
            plsc.store_scatter(dst, [p2, q2 + w], wi)


def _phase_a(n_rows, tabt_hbm, gamma_hbm, beta_hbm, tail_hbm, scr_hbm,
             in0_v, in1_v, out0_v, out1_v, gam_v, bet_v,
             gsem0, gsem1, ssem0, ssem1):
    cid = lax.axis_index("c")
    sid = lax.axis_index("s")
    wid = sid * _NC + cid

    pltpu.sync_copy(gamma_hbm, gam_v)
    pltpu.sync_copy(beta_hbm, bet_v)
    ga, be = [], []
    for k in range(_CH // _L):
        gk = gam_v[pl.ds(k * _L, _L)]
        bk = bet_v[pl.ds(k * _L, _L)]
        for l in range(_L):
            ga.append(gk[l])
            be.append(bk[l])

    n_full = n_rows // _ACOL
    _NB = 2
    n_iter = -((n_full + _NW - 1) // _NW // -_NB) * _NB
    inb = (in0_v, in1_v)
    outb = (out0_v, out1_v)
    gsem = (gsem0, gsem1)
    ssem = (ssem0, ssem1)

    def blk_of(i):
        return wid + i * _NW

    def start_in(i, b):
        pltpu.make_async_copy(
            tabt_hbm.at[:, pl.ds(blk_of(i) * _ACOL, _ACOL)], inb[b],
            gsem[b]).start()

    def wait_in(b):
        pltpu.make_async_copy(
            tabt_hbm.at[:, pl.ds(0, _ACOL)], inb[b], gsem[b]).wait()

    def start_out(i, b):
        pltpu.make_async_copy(
            outb[b], scr_hbm.at[pl.ds(blk_of(i) * (_ACOL // 4), _ACOL // 4)],
            ssem[b]).start()

    def wait_out(b):
        pltpu.make_async_copy(
            outb[b], scr_hbm.at[pl.ds(0, _ACOL // 4)], ssem[b]).wait()

    for b in range(_NB):
        start_in(b, b)

    @pl.loop(0, n_iter, step=_NB)
    def _(ii):
        for b in range(_NB):
            i = ii + b

            @pl.when(blk_of(i) < n_full)
            def _():
                wait_in(b)

                @pl.when(i >= _NB)
                def _():
                    wait_out(b)

                _norm_block(inb[b], outb[b], _ACOL, ga, be)
                start_out(i, b)

                @pl.when(blk_of(i + _NB) < n_full)
                def _():
                    start_in(i + _NB, b)

    for b in range(_NB):
        wait_out(b)

    tail = n_rows - n_full * _ACOL
    if tail:
        @pl.when(wid == _NW - 1)
        def _():
            pltpu.sync_copy(tail_hbm, in0_v.at[:, pl.ds(0, _BLK)])
            _norm_block(in0_v, out0_v, tail, ga, be)
            pltpu.sync_copy(
                out0_v.at[pl.ds(0, tail // 4)],
                scr_hbm.at[pl.ds(n_full * (_ACOL // 4), tail // 4)])


def _phase_b(n_m, xt_hbm, scr_hbm, out_hbm,
             idx_v, in0_v, in1_v, in2_v, in3_v, tr0_v, tr1_v,
             gsem0, gsem1, gsem2, gsem3, ssem0, ssem1):
    cid = lax.axis_index("c")
    sid = lax.axis_index("s")
    wid = sid * _NC + cid

    pltpu.sync_copy(xt_hbm.at[:, pl.ds(wid * _BB, _BB)], idx_v)

    inb = (in0_v, in1_v, in2_v, in3_v)
    trb = (tr0_v, tr1_v)
    gsem = (gsem0, gsem1, gsem2, gsem3)
    ssem = (ssem0, ssem1)
    _NB = 4

    def start_gather(m, b):
        pltpu.make_async_copy(
            scr_hbm.at[idx_v.at[m]], inb[b], gsem[b]).start()

    def wait_gather(b):
        pltpu.make_async_copy(
            scr_hbm.at[idx_v.at[0]], inb[b], gsem[b]).wait()

    def start_store(m, b):
        pltpu.make_async_copy(trb[b], out_hbm.at[m, :, wid], ssem[b]).start()

    def wait_store(b):
        pltpu.make_async_copy(trb[b], out_hbm.at[0, :, wid], ssem[b]).wait()

    for b in range(_NB):
        start_gather(b, b)

    @pl.loop(0, n_m, step=_NB)
    def _(mm):
        for b in range(_NB):
            m = mm + b
            wait_gather(b)

            @pl.when(m >= 2)
            def _():
                wait_store(b % 2)

            src, dst = inb[b], trb[b % 2]

            @plsc.parallel_loop(0, _BB, _L)
            def _(r0):
                rows = r0 + lax.iota(jnp.int32, _L)
                for w0 in range(0, 32, 8):
                    vs = [plsc.load_gather(
                              src, [rows, jnp.full((_L,), w0 + j, jnp.int32)])
                          for j in range(8)]
                    for j in range(8):
                        w = w0 + j
                        lo, hi = plsc.unpack(
                            plsc.bitcast(vs[j], jnp.bfloat16),
                            format=plsc.PackFormat.INTERLEAVED)
                        c1 = w + (0 if w < _L else _L)
                        c2 = c1 + _L
                        dst[c1 // 8, c1 % 8, pl.ds(r0, _L)] = lo
                        dst[c2 // 8, c2 % 8, pl.ds(r0, _L)] = hi

            start_store(m, b % 2)

            @pl.when(m + _NB < n_m)
            def _():
                start_gather(m + _NB, b)

    wait_store(0)
    wait_store(1)


def kernel(x, table, gamma, beta):
    n_b, n_m = x.shape
    n_rows = table.shape[0]
    xt = x.T
    tabt = table.T
    n_full = n_rows // _BLK
    tail_w = n_rows - n_full * _BLK
    tail = jnp.pad(table[n_full * _BLK:].T, ((0, 0), (0, _BLK - tail_w)))

    mesh = plsc.VectorSubcoreMesh(
        core_axis_name="c", subcore_axis_name="s",
        num_cores=_NC, num_subcores=_NS)

    norm_tab = pl.kernel(
        functools.partial(_phase_a, n_rows),
        out_type=jax.ShapeDtypeStruct((n_rows // 4, _BB), jnp.int32),
        mesh=mesh,
        compiler_params=pltpu.CompilerParams(needs_layout_passes=False),
        scratch_types=(
            [pltpu.VMEM((_CH, _ACOL), jnp.float32)] * 2
            + [pltpu.VMEM((_ACOL // 4, _BB), jnp.int32)] * 2
            + [pltpu.VMEM((_CH,), jnp.float32)] * 2
            + [pltpu.SemaphoreType.DMA] * 4
        ),
    )(tabt, gamma, beta, tail)

    scr = norm_tab.reshape(n_rows, 32)

    out5d = pl.kernel(
        functools.partial(_phase_b, n_m),
        out_type=jax.ShapeDtypeStruct((n_m, 8, _NW, 8, _BB), jnp.float32),
        mesh=mesh,
        compiler_params=pltpu.CompilerParams(
            needs_layout_passes=False, use_tc_tiling_on_sc=False),
        scratch_types=(
            [pltpu.VMEM((n_m, _BB), jnp.int32)]
            + [pltpu.VMEM((_BB, 32), jnp.int32)] * 4
            + [pltpu.VMEM((8, 8, _BB), jnp.float32)] * 2
            + [pltpu.SemaphoreType.DMA] * 6
        ),
    )(xt, scr)

    out = out5d.transpose(2, 4, 0, 1, 3).reshape(n_b, n_m, _CH)
    return out
